# baseline (speedup 1.0000x reference)
import functools

import jax
import jax.numpy as jnp
from jax import lax
from jax.experimental import pallas as pl
from jax.experimental.pallas import tpu as pltpu

_MARGIN = 0.2


def _fused_kernel(im_ref, s_ref, lens_ref, out_ref,
                  pim_s, ps_s, *, t_full, tb, nb, margin):
    i = pl.program_id(0)

    @pl.when(i < nb)
    def _pool():
        im = im_ref[...]
        s = s_ref[...]
        r = im.shape[0]
        t = s.shape[0]

        lens = lens_ref[...]
        nc = lens.shape[1]
        row0 = (i * tb) % 128
        col0 = (i * tb) // 128
        colsel = (lax.broadcasted_iota(jnp.int32, (nc, 1), 0)
                  == col0).astype(jnp.float32)
        im_row = jnp.sum(lens[0] * colsel, axis=0, keepdims=True)
        s_row = jnp.sum(lens[1] * colsel, axis=0, keepdims=True)
        lanesel = (lax.broadcasted_iota(jnp.int32, (tb, 128), 1)
                   == row0 + lax.broadcasted_iota(jnp.int32, (tb, 128), 0)
                   ).astype(jnp.float32)
        im_len = jnp.sum(lanesel * im_row, axis=1, keepdims=True)[None]
        s_len = jnp.sum(lanesel * s_row, axis=1, keepdims=True)[None]

        r_idx = lax.broadcasted_iota(jnp.int32, (r, tb, 1), 0
                                     ).astype(jnp.float32)
        t_idx = lax.broadcasted_iota(jnp.int32, (t, tb, 1), 0
                                     ).astype(jnp.float32)
        im_mask = ((r_idx >= 1.0) & (r_idx < im_len)).astype(jnp.float32)
        s_mask = ((t_idx >= 1.0) & (t_idx <= t_full - 3.0)
                  & (t_idx < s_len - 2.0)).astype(jnp.float32)

        pim = jnp.sum(im * im_mask, axis=0)
        ps = jnp.sum(s * s_mask, axis=0)
        pim_s[pl.ds(i * tb, tb), :] = pim.astype(jnp.bfloat16)
        ps_s[pl.ds(i * tb, tb), :] = ps.astype(jnp.bfloat16)

    @pl.when(i == nb)
    def _loss():
        scores = lax.dot_general(
            pim_s[...], ps_s[...],
            dimension_numbers=(((1,), (1,)), ((), ())),
            preferred_element_type=jnp.float32)
        b = scores.shape[0]
        row = lax.broadcasted_iota(jnp.int32, (b, b), 0)
        col = lax.broadcasted_iota(jnp.int32, (b, b), 1)
        eye = row == col

        diag = jnp.where(eye, scores, 0.0)
        d1 = jnp.sum(diag, axis=1, keepdims=True)
        d2 = jnp.sum(diag, axis=0, keepdims=True)

        cost_s = jnp.where(eye, 0.0, jnp.maximum(margin + scores - d1, 0.0))
        cost_im = jnp.where(eye, 0.0, jnp.maximum(margin + scores - d2, 0.0))
        total = jnp.sum(cost_s) + jnp.sum(cost_im)
        out_ref[...] = total.reshape(1, 1)


def kernel(im_set, s_seq, im_len, s_len):
    im_set = jnp.asarray(im_set)
    s_seq = jnp.asarray(s_seq)
    b, r, d = im_set.shape
    b_s, t, _ = s_seq.shape
    assert b == b_s, "contrastive loss requires a square score matrix"
    assert b % 128 == 0, "batch must be lane-tileable"
    lens = jnp.stack([jnp.asarray(im_len, jnp.int32),
                      jnp.asarray(s_len, jnp.int32)]
                     ).astype(jnp.float32).reshape(2, b // 128, 128)
    im_t = jnp.transpose(im_set, (1, 0, 2))
    s_t = jnp.transpose(s_seq, (1, 0, 2))

    tb = 32
    while b % tb:
        tb //= 2
    nb = b // tb
    clamp = nb - 1

    out = pl.pallas_call(
        functools.partial(_fused_kernel, t_full=t, tb=tb, nb=nb,
                          margin=_MARGIN),
        out_shape=jax.ShapeDtypeStruct((1, 1), jnp.float32),
        grid=(nb + 1,),
        in_specs=[
            pl.BlockSpec((r, tb, d), lambda i: (0, jnp.minimum(i, clamp), 0)),
            pl.BlockSpec((t, tb, d), lambda i: (0, jnp.minimum(i, clamp), 0)),
            pl.BlockSpec((2, b // 128, 128), lambda i: (0, 0, 0)),
        ],
        out_specs=pl.BlockSpec((1, 1), lambda i: (0, 0)),
        scratch_shapes=[pltpu.VMEM((b, d), jnp.bfloat16),
                        pltpu.VMEM((b, d), jnp.bfloat16)],
        compiler_params=pltpu.CompilerParams(
            dimension_semantics=("arbitrary",),
            vmem_limit_bytes=60 << 20),
    )(im_t, s_t, lens)
    return out[0, 0]


# speedup vs baseline: 1.0347x; 1.0347x over previous
import functools

import jax
import jax.numpy as jnp
from jax import lax
from jax.experimental import pallas as pl
from jax.experimental.pallas import tpu as pltpu

_MARGIN = 0.2


def _fused_kernel(im_ref, s_ref, im_len_ref, s_len_ref, out_ref,
                  pim_s, ps_s, ilen_s, slen_s, *, t_full, tb, nb, margin):
    i = pl.program_id(0)

    @pl.when(i == 0)
    def _unpack_lens():
        eye = (lax.broadcasted_iota(jnp.int32, (128, 128), 0)
               == lax.broadcasted_iota(jnp.int32, (128, 128), 1))
        ilen = im_len_ref[...]
        slen = s_len_ref[...]
        for c in range(ilen.shape[0]):
            ilen_s[c * 128:(c + 1) * 128, :] = jnp.sum(
                jnp.where(eye, ilen[c:c + 1, :], 0), axis=1, keepdims=True)
            slen_s[c * 128:(c + 1) * 128, :] = jnp.sum(
                jnp.where(eye, slen[c:c + 1, :], 0), axis=1, keepdims=True)

    @pl.when(i < nb)
    def _pool():
        im = im_ref[...]
        s = s_ref[...]
        r = im.shape[0]
        t = s.shape[0]

        im_len = ilen_s[pl.ds(i * tb, tb), :][None]
        s_len = slen_s[pl.ds(i * tb, tb), :][None]
        r_idx = lax.broadcasted_iota(jnp.int32, (r, tb, 1), 0)
        t_idx = lax.broadcasted_iota(jnp.int32, (t, tb, 1), 0)
        im_mask = ((r_idx >= 1) & (r_idx < im_len)).astype(jnp.float32)
        s_mask = ((t_idx >= 1) & (t_idx <= t_full - 3)
                  & (t_idx < s_len - 2)).astype(jnp.float32)

        pim = jnp.sum(im * im_mask, axis=0)
        ps = jnp.sum(s * s_mask, axis=0)
        pim_s[pl.ds(i * tb, tb), :] = pim.astype(jnp.bfloat16)
        ps_s[pl.ds(i * tb, tb), :] = ps.astype(jnp.bfloat16)

    @pl.when(i == nb)
    def _loss():
        scores = lax.dot_general(
            pim_s[...], ps_s[...],
            dimension_numbers=(((1,), (1,)), ((), ())),
            preferred_element_type=jnp.float32)
        b = scores.shape[0]
        row = lax.broadcasted_iota(jnp.int32, (b, b), 0)
        col = lax.broadcasted_iota(jnp.int32, (b, b), 1)
        eye = row == col

        diag = jnp.where(eye, scores, 0.0)
        d1 = jnp.sum(diag, axis=1, keepdims=True)
        d2 = jnp.sum(diag, axis=0, keepdims=True)

        cost_s = jnp.where(eye, 0.0, jnp.maximum(margin + scores - d1, 0.0))
        cost_im = jnp.where(eye, 0.0, jnp.maximum(margin + scores - d2, 0.0))
        total = jnp.sum(cost_s) + jnp.sum(cost_im)
        out_ref[...] = total.reshape(1, 1)


def kernel(im_set, s_seq, im_len, s_len):
    im_set = jnp.asarray(im_set)
    s_seq = jnp.asarray(s_seq)
    b, r, d = im_set.shape
    b_s, t, _ = s_seq.shape
    assert b == b_s, "contrastive loss requires a square score matrix"
    assert b % 128 == 0, "batch must be lane-tileable"
    im_len2 = jnp.asarray(im_len, jnp.int32).reshape(b // 128, 128)
    s_len2 = jnp.asarray(s_len, jnp.int32).reshape(b_s // 128, 128)
    im_t = jnp.transpose(im_set, (1, 0, 2))
    s_t = jnp.transpose(s_seq, (1, 0, 2))

    tb = 32
    while b % tb:
        tb //= 2
    nb = b // tb
    clamp = nb - 1

    out = pl.pallas_call(
        functools.partial(_fused_kernel, t_full=t, tb=tb, nb=nb,
                          margin=_MARGIN),
        out_shape=jax.ShapeDtypeStruct((1, 1), jnp.float32),
        grid=(nb + 1,),
        in_specs=[
            pl.BlockSpec((r, tb, d), lambda i: (0, jnp.minimum(i, clamp), 0)),
            pl.BlockSpec((t, tb, d), lambda i: (0, jnp.minimum(i, clamp), 0)),
            pl.BlockSpec((b // 128, 128), lambda i: (0, 0)),
            pl.BlockSpec((b // 128, 128), lambda i: (0, 0)),
        ],
        out_specs=pl.BlockSpec((1, 1), lambda i: (0, 0)),
        scratch_shapes=[pltpu.VMEM((b, d), jnp.bfloat16),
                        pltpu.VMEM((b, d), jnp.bfloat16),
                        pltpu.VMEM((b, 1), jnp.int32),
                        pltpu.VMEM((b, 1), jnp.int32)],
        compiler_params=pltpu.CompilerParams(
            dimension_semantics=("arbitrary",),
            vmem_limit_bytes=60 << 20),
    )(im_t, s_t, im_len2, s_len2)
    return out[0, 0]
